# initial kernel scaffold (unmeasured)
import jax
import jax.numpy as jnp
from jax import lax
from jax.experimental import pallas as pl
from jax.experimental.pallas import tpu as pltpu


def kernel(x, dy):
    k_per, d = x.shape
    _, f = dy.shape
    half = d // 2

    def body(x_ref, dy_ref, out_ref, send_buf, recv_buf, send_sem, recv_sem):
        ix = lax.axis_index("x")
        iy = lax.axis_index("y")
        iz = lax.axis_index("z")
        px = 1 - ix

        barrier_sem = pltpu.get_barrier_semaphore()
        pl.semaphore_signal(
            barrier_sem, inc=1,
            device_id=(px, iy, iz), device_id_type=pl.DeviceIdType.MESH,
        )
        pl.semaphore_wait(barrier_sem, 1)

        dyb = dy_ref[...].astype(jnp.bfloat16)

        x_p = x_ref[:, pl.ds(px * half, half)].astype(jnp.bfloat16)
        partial_p = lax.dot_general(
            x_p, dyb,
            dimension_numbers=(((0,), (0,)), ((), ())),
            preferred_element_type=jnp.float32,
        )
        send_buf[...] = partial_p.astype(jnp.bfloat16)

        rdma = pltpu.make_async_remote_copy(
            src_ref=send_buf,
            dst_ref=recv_buf,
            send_sem=send_sem,
            recv_sem=recv_sem,
            device_id=(px, iy, iz),
            device_id_type=pl.DeviceIdType.MESH,
        )
        rdma.start()

        x_m = x_ref[:, pl.ds(ix * half, half)].astype(jnp.bfloat16)
        partial_m = lax.dot_general(
            x_m, dyb,
            dimension_numbers=(((0,), (0,)), ((), ())),
            preferred_element_type=jnp.float32,
        )

        rdma.wait()
        out_ref[...] = partial_m + recv_buf[...].astype(jnp.float32)

    return pl.pallas_call(
        body,
        out_shape=jax.ShapeDtypeStruct((half, f), jnp.float32),
        in_specs=[
            pl.BlockSpec(memory_space=pltpu.VMEM),
            pl.BlockSpec(memory_space=pltpu.VMEM),
        ],
        out_specs=pl.BlockSpec(memory_space=pltpu.VMEM),
        scratch_shapes=[
            pltpu.VMEM((half, f), jnp.bfloat16),
            pltpu.VMEM((half, f), jnp.bfloat16),
            pltpu.SemaphoreType.DMA,
            pltpu.SemaphoreType.DMA,
        ],
        compiler_params=pltpu.CompilerParams(collective_id=0),
    )(x, dy)


# baseline (device time: 74874 ns/iter reference)
import jax
import jax.numpy as jnp
from jax import lax
from jax.experimental import pallas as pl
from jax.experimental.pallas import tpu as pltpu


def kernel(x, dy):
    k_per, d = x.shape
    _, f = dy.shape
    half = d // 2

    def body(x_ref, dy_ref, out_ref, send_buf, recv_buf, send_sem, recv_sem):
        ix = lax.axis_index("x")
        iy = lax.axis_index("y")
        iz = lax.axis_index("z")
        px = 1 - ix

        barrier_sem = pltpu.get_barrier_semaphore()
        pl.semaphore_signal(
            barrier_sem, inc=1,
            device_id=(px, iy, iz), device_id_type=pl.DeviceIdType.MESH,
        )
        pl.semaphore_wait(barrier_sem, 1)

        dyb = dy_ref[...].astype(jnp.bfloat16)

        x_p = x_ref[:, pl.ds(px * half, half)].astype(jnp.bfloat16)
        partial_p = lax.dot_general(
            x_p, dyb,
            dimension_numbers=(((0,), (0,)), ((), ())),
            preferred_element_type=jnp.float32,
        )
        send_buf[...] = partial_p.astype(jnp.bfloat16)

        rdma = pltpu.make_async_remote_copy(
            src_ref=send_buf,
            dst_ref=recv_buf,
            send_sem=send_sem,
            recv_sem=recv_sem,
            device_id=(px, iy, iz),
            device_id_type=pl.DeviceIdType.MESH,
        )
        rdma.start()

        x_m = x_ref[:, pl.ds(ix * half, half)].astype(jnp.bfloat16)
        partial_m = lax.dot_general(
            x_m, dyb,
            dimension_numbers=(((0,), (0,)), ((), ())),
            preferred_element_type=jnp.float32,
        )

        rdma.wait()
        out_ref[...] = partial_m + recv_buf[...].astype(jnp.float32)

    return pl.pallas_call(
        body,
        out_shape=jax.ShapeDtypeStruct((half, f), jnp.float32),
        in_specs=[
            pl.BlockSpec(memory_space=pltpu.VMEM),
            pl.BlockSpec(memory_space=pltpu.VMEM),
        ],
        out_specs=pl.BlockSpec(memory_space=pltpu.VMEM),
        scratch_shapes=[
            pltpu.VMEM((half, f), jnp.bfloat16),
            pltpu.VMEM((half, f), jnp.bfloat16),
            pltpu.SemaphoreType.DMA,
            pltpu.SemaphoreType.DMA,
        ],
        compiler_params=pltpu.CompilerParams(
            collective_id=0, vmem_limit_bytes=100 * 1024 * 1024
        ),
    )(x, dy)


# device time: 66262 ns/iter; 1.1300x vs baseline; 1.1300x over previous
import jax
import jax.numpy as jnp
from jax import lax
from jax.experimental import pallas as pl
from jax.experimental.pallas import tpu as pltpu

NC = 8


def kernel(x, dy):
    k_per, d = x.shape
    _, f = dy.shape
    half = d // 2
    fc = f // NC

    def body(x_ref, dy_ref, out_ref, send_buf, recv_buf, send_sems, recv_sems):
        ix = lax.axis_index("x")
        iy = lax.axis_index("y")
        iz = lax.axis_index("z")
        px = 1 - ix

        barrier_sem = pltpu.get_barrier_semaphore()
        pl.semaphore_signal(
            barrier_sem, inc=1,
            device_id=(px, iy, iz), device_id_type=pl.DeviceIdType.MESH,
        )
        pl.semaphore_wait(barrier_sem, 1)

        x_p = x_ref[:, pl.ds(px * half, half)].astype(jnp.bfloat16)
        x_m = x_ref[:, pl.ds(ix * half, half)].astype(jnp.bfloat16)

        dims = (((0,), (0,)), ((), ()))
        rdmas = []
        for j in range(NC):
            cols = pl.ds(j * fc, fc)
            dyb_j = dy_ref[:, cols].astype(jnp.bfloat16)
            pp_j = lax.dot_general(
                x_p, dyb_j, dims, preferred_element_type=jnp.float32
            )
            send_buf[:, cols] = pp_j.astype(jnp.bfloat16)
            rdma = pltpu.make_async_remote_copy(
                src_ref=send_buf.at[:, cols],
                dst_ref=recv_buf.at[:, cols],
                send_sem=send_sems.at[j],
                recv_sem=recv_sems.at[j],
                device_id=(px, iy, iz),
                device_id_type=pl.DeviceIdType.MESH,
            )
            rdma.start()
            rdmas.append(rdma)
            pm_j = lax.dot_general(
                x_m, dyb_j, dims, preferred_element_type=jnp.float32
            )
            out_ref[:, cols] = pm_j

        for j in range(NC):
            cols = pl.ds(j * fc, fc)
            rdmas[j].wait()
            out_ref[:, cols] = out_ref[:, cols] + recv_buf[:, cols].astype(
                jnp.float32
            )

    return pl.pallas_call(
        body,
        out_shape=jax.ShapeDtypeStruct((half, f), jnp.float32),
        in_specs=[
            pl.BlockSpec(memory_space=pltpu.VMEM),
            pl.BlockSpec(memory_space=pltpu.VMEM),
        ],
        out_specs=pl.BlockSpec(memory_space=pltpu.VMEM),
        scratch_shapes=[
            pltpu.VMEM((half, f), jnp.bfloat16),
            pltpu.VMEM((half, f), jnp.bfloat16),
            pltpu.SemaphoreType.DMA((NC,)),
            pltpu.SemaphoreType.DMA((NC,)),
        ],
        compiler_params=pltpu.CompilerParams(
            collective_id=0, vmem_limit_bytes=100 * 1024 * 1024
        ),
    )(x, dy)


# device time: 49618 ns/iter; 1.5090x vs baseline; 1.3354x over previous
import jax
import jax.numpy as jnp
from jax import lax
from jax.experimental import pallas as pl
from jax.experimental.pallas import tpu as pltpu

NC = 8


def kernel(x, dy):
    k_per, d = x.shape
    _, f = dy.shape
    half = d // 2
    fhalf = f // 2
    fc = fhalf // NC

    def body(
        x_ref, dy_ref, out_ref,
        send_x, recv_x, send_y, recv_y,
        sx_sems, rx_sems, sy_sems, ry_sems,
    ):
        ix = lax.axis_index("x")
        iy = lax.axis_index("y")
        iz = lax.axis_index("z")
        px = 1 - ix
        py = iy ^ 1
        h = iy & 1

        barrier_sem = pltpu.get_barrier_semaphore()
        pl.semaphore_signal(
            barrier_sem, inc=1,
            device_id=(px, iy, iz), device_id_type=pl.DeviceIdType.MESH,
        )
        pl.semaphore_signal(
            barrier_sem, inc=1,
            device_id=(ix, py, iz), device_id_type=pl.DeviceIdType.MESH,
        )
        pl.semaphore_wait(barrier_sem, 2)

        x_p = x_ref[:, pl.ds(px * half, half)].astype(jnp.bfloat16)
        x_m = x_ref[:, pl.ds(ix * half, half)].astype(jnp.bfloat16)

        dims = (((0,), (0,)), ((), ()))
        my_base = h * fhalf
        other_base = (1 - h) * fhalf

        x_rdmas = []
        for j in range(NC):
            jcols = pl.ds(j * fc, fc)
            mycols = pl.ds(my_base + j * fc, fc)
            dyb_j = dy_ref[:, mycols].astype(jnp.bfloat16)
            pp_j = lax.dot_general(
                x_p, dyb_j, dims, preferred_element_type=jnp.float32
            )
            send_x[:, jcols] = pp_j.astype(jnp.bfloat16)
            rdma = pltpu.make_async_remote_copy(
                src_ref=send_x.at[:, jcols],
                dst_ref=recv_x.at[:, jcols],
                send_sem=sx_sems.at[j],
                recv_sem=rx_sems.at[j],
                device_id=(px, iy, iz),
                device_id_type=pl.DeviceIdType.MESH,
            )
            rdma.start()
            x_rdmas.append(rdma)
            pm_j = lax.dot_general(
                x_m, dyb_j, dims, preferred_element_type=jnp.float32
            )
            out_ref[:, mycols] = pm_j

        y_rdmas = []
        for j in range(NC):
            jcols = pl.ds(j * fc, fc)
            mycols = pl.ds(my_base + j * fc, fc)
            x_rdmas[j].wait()
            r_j = out_ref[:, mycols] + recv_x[:, jcols].astype(jnp.float32)
            out_ref[:, mycols] = r_j
            send_y[:, jcols] = r_j.astype(jnp.bfloat16)
            rdma = pltpu.make_async_remote_copy(
                src_ref=send_y.at[:, jcols],
                dst_ref=recv_y.at[:, jcols],
                send_sem=sy_sems.at[j],
                recv_sem=ry_sems.at[j],
                device_id=(ix, py, iz),
                device_id_type=pl.DeviceIdType.MESH,
            )
            rdma.start()
            y_rdmas.append(rdma)

        for j in range(NC):
            jcols = pl.ds(j * fc, fc)
            othercols = pl.ds(other_base + j * fc, fc)
            y_rdmas[j].wait()
            out_ref[:, othercols] = recv_y[:, jcols].astype(jnp.float32)

    buf = pltpu.VMEM((half, fhalf), jnp.bfloat16)
    return pl.pallas_call(
        body,
        out_shape=jax.ShapeDtypeStruct((half, f), jnp.float32),
        in_specs=[
            pl.BlockSpec(memory_space=pltpu.VMEM),
            pl.BlockSpec(memory_space=pltpu.VMEM),
        ],
        out_specs=pl.BlockSpec(memory_space=pltpu.VMEM),
        scratch_shapes=[
            buf, buf, buf, buf,
            pltpu.SemaphoreType.DMA((NC,)),
            pltpu.SemaphoreType.DMA((NC,)),
            pltpu.SemaphoreType.DMA((NC,)),
            pltpu.SemaphoreType.DMA((NC,)),
        ],
        compiler_params=pltpu.CompilerParams(
            collective_id=0, vmem_limit_bytes=100 * 1024 * 1024
        ),
    )(x, dy)
